# baseline (speedup 1.0000x reference)
import functools

import jax
import jax.numpy as jnp
import numpy as np
from jax import lax
from jax.experimental import pallas as pl
from jax.experimental.pallas import tpu as pltpu
from jax.experimental.pallas import tpu_sc as plsc

HID = 2048
NHEAD = 16
HDIM = HID // NHEAD
SPARSITY = 0.99609375
TPB = 128 // NHEAD


def _scores_kernel(x_ref, wq_ref, bq_ref, wk_ref, bk_ref, s_ref):
    t = x_ref.shape[0]
    g = t // TPB
    xb = x_ref[...].astype(jnp.bfloat16)
    q = jnp.dot(xb, wq_ref[...], preferred_element_type=jnp.float32) + bq_ref[...]
    k = jnp.dot(xb, wk_ref[...], preferred_element_type=jnp.float32) + bk_ref[...]
    qr = q.reshape(g, 128, 128).astype(jnp.bfloat16)
    kr = k.reshape(g, 128, 128).astype(jnp.bfloat16)
    sfull = jax.lax.dot_general(
        qr, kr, (((2,), (2,)), ((0,), (0,))),
        preferred_element_type=jnp.float32)
    s_ref[...] = sfull * np.float32(1.0 / np.sqrt(HDIM))


def _av_out_kernel(*refs):
    x_ref, wv_ref, bv_ref, wo_ref, bo_ref, bd_ref, o_ref = refs[-7:]
    t = x_ref.shape[0]
    g = t // TPB
    xb = x_ref[...].astype(jnp.bfloat16)
    v = jnp.dot(xb, wv_ref[...], preferred_element_type=jnp.float32) + bv_ref[...]
    vr = v.reshape(g, 128, 128).astype(jnp.bfloat16)
    attn = jax.lax.dot_general(
        bd_ref[...].astype(jnp.bfloat16), vr, (((2,), (1,)), ((0,), (0,))),
        preferred_element_type=jnp.float32)
    attn = attn.reshape(t, HID).astype(jnp.bfloat16)
    o_ref[...] = (jnp.dot(attn, wo_ref[...], preferred_element_type=jnp.float32)
                  + bo_ref[...])


def _sc_mask_softmax(sfull, topk):
    gtot = sfull.shape[0]
    info = plsc.get_sparse_core_info()
    nc, ns = info.num_cores, info.num_subcores
    nw = nc * ns
    g_per_w = gtot // nw
    mesh = plsc.VectorSubcoreMesh(core_axis_name="c", subcore_axis_name="s")

    @functools.partial(
        pl.kernel, mesh=mesh,
        out_type=jax.ShapeDtypeStruct((gtot, 128, 128), jnp.float32),
        scratch_types=[
            pltpu.VMEM((128, 128), jnp.float32),
            pltpu.VMEM((128, 128), jnp.float32),
            pltpu.SemaphoreType.DMA,
        ],
        compiler_params=pltpu.CompilerParams(needs_layout_passes=False),
    )
    def sc_kernel(sf_hbm, out_hbm, in_tile, out_tile, sem):
        wid = lax.axis_index("s") * nc + lax.axis_index("c")
        base = wid * g_per_w
        zeros16 = jnp.zeros((NHEAD,), jnp.float32)

        @pl.loop(0, 128)
        def _zero_row(r):
            @pl.loop(0, 128, step=NHEAD)
            def _zero_chunk(c):
                out_tile.at[r].at[pl.ds(c, NHEAD)][...] = zeros16

        @pl.loop(0, g_per_w)
        def _per_tile(gi):
            g = base + gi
            pltpu.async_copy(sf_hbm.at[g], in_tile, sem).wait()
            for a in range(TPB):
                @pl.loop(0, NHEAD)
                def _per_row(h, a=a):
                    s = in_tile.at[NHEAD * a + h].at[pl.ds(NHEAD * a, NHEAD)][...]
                    srt = lax.sort(s, dimension=0)
                    pos = lax.iota(jnp.int32, NHEAD)
                    thr = jnp.max(jnp.where(pos == NHEAD - topk, srt,
                                            -jnp.inf))
                    gt = s > thr
                    eq = s == thr
                    cnt_gt = jnp.sum(jnp.where(gt, 1.0, 0.0))
                    eqf = jnp.where(eq, 1.0, 0.0)
                    prefix_eq = jnp.cumsum(eqf) - eqf
                    keep = gt | (eq & (prefix_eq < (topk - cnt_gt)))
                    sp = jnp.where(keep, s, 0.0)
                    m = jnp.max(sp)
                    e = jnp.exp(sp - m)
                    w = e / jnp.sum(e)
                    out_tile.at[NHEAD * a + h].at[pl.ds(NHEAD * a, NHEAD)][...] = w

            pltpu.sync_copy(out_tile, out_hbm.at[g])

    return sc_kernel(sfull)


def _k1_scores(xf, Wq, bq2, Wk, bk2, t1, csz, c):
    n, d = xf.shape
    off = c * (csz // t1)
    return pl.pallas_call(
        _scores_kernel,
        grid=(csz // t1,),
        in_specs=[
            pl.BlockSpec((t1, d), lambda i: (i + off, 0)),
            pl.BlockSpec((d, d), lambda i: (0, 0)),
            pl.BlockSpec((1, d), lambda i: (0, 0)),
            pl.BlockSpec((d, d), lambda i: (0, 0)),
            pl.BlockSpec((1, d), lambda i: (0, 0)),
        ],
        out_specs=pl.BlockSpec((t1 // TPB, 128, 128), lambda i: (i, 0, 0)),
        out_shape=jax.ShapeDtypeStruct((csz // TPB, 128, 128), jnp.float32),
        compiler_params=pltpu.CompilerParams(
            dimension_semantics=("arbitrary",)),
    )(xf, Wq, bq2, Wk, bk2)


def _k2_output(obuf, xf, Wv, bv2, Wo, bo2, bdw, t2, csz, c):
    n, d = xf.shape
    off = c * (csz // t2)
    specs = [
        pl.BlockSpec((t2, d), lambda i: (i + off, 0)),
        pl.BlockSpec((d, d), lambda i: (0, 0)),
        pl.BlockSpec((1, d), lambda i: (0, 0)),
        pl.BlockSpec((d, d), lambda i: (0, 0)),
        pl.BlockSpec((1, d), lambda i: (0, 0)),
        pl.BlockSpec((t2 // TPB, 128, 128), lambda i: (i, 0, 0)),
    ]
    args = [xf, Wv, bv2, Wo, bo2, bdw]
    aliases = {}
    if obuf is not None:
        specs = [pl.BlockSpec(memory_space=pl.ANY)] + specs
        args = [obuf] + args
        aliases = {0: 0}
    return pl.pallas_call(
        _av_out_kernel,
        grid=(csz // t2,),
        in_specs=specs,
        out_specs=pl.BlockSpec((t2, d), lambda i: (i + off, 0)),
        out_shape=jax.ShapeDtypeStruct((n, d), jnp.float32),
        input_output_aliases=aliases,
        compiler_params=pltpu.CompilerParams(
            dimension_semantics=("arbitrary",)),
    )(*args)


@functools.partial(jax.jit, static_argnames=())
def kernel(x, Wq, bq, Wk, bk, Wv, bv, Wo, bo):
    b, s, d = x.shape
    n = b * s
    xf = x.reshape(n, d)
    topk = int(s * (1.0 - SPARSITY))
    t1 = 512
    t2 = 256
    nchunks = 2
    csz = n // nchunks

    bq2, bk2 = bq.reshape(1, d), bk.reshape(1, d)
    bv2, bo2 = bv.reshape(1, d), bo.reshape(1, d)
    Wqh, Wkh = Wq.astype(jnp.bfloat16), Wk.astype(jnp.bfloat16)
    Wvh, Woh = Wv.astype(jnp.bfloat16), Wo.astype(jnp.bfloat16)

    bdws = []
    for c in range(nchunks):
        sfull = _k1_scores(xf, Wqh, bq2, Wkh, bk2, t1, csz, c)
        bdws.append(_sc_mask_softmax(sfull, topk))
    out = None
    for c in range(nchunks):
        out = _k2_output(out, xf, Wvh, bv2, Woh, bo2, bdws[c], t2, csz, c)

    return out.reshape(b, s, d)


# speedup vs baseline: 1.0429x; 1.0429x over previous
import functools

import jax
import jax.numpy as jnp
import numpy as np
from jax import lax
from jax.experimental import pallas as pl
from jax.experimental.pallas import tpu as pltpu
from jax.experimental.pallas import tpu_sc as plsc

HID = 2048
NHEAD = 16
HDIM = HID // NHEAD
SPARSITY = 0.99609375
TPB = 128 // NHEAD


def _scores_kernel(x_ref, wq_ref, bq_ref, wk_ref, bk_ref, s_ref):
    t = x_ref.shape[0]
    g = t // TPB
    xb = x_ref[...]
    q = jnp.dot(xb, wq_ref[...], preferred_element_type=jnp.float32) + bq_ref[...]
    k = jnp.dot(xb, wk_ref[...], preferred_element_type=jnp.float32) + bk_ref[...]
    qr = q.reshape(g, 128, 128)
    kr = k.reshape(g, 128, 128)
    sfull = jax.lax.dot_general(
        qr, kr, (((2,), (2,)), ((0,), (0,))),
        preferred_element_type=jnp.float32)
    s_ref[...] = sfull * np.float32(1.0 / np.sqrt(HDIM))


def _av_out_kernel(*refs):
    x_ref, wv_ref, bv_ref, wo_ref, bo_ref, bd_ref, o_ref = refs[-7:]
    t = x_ref.shape[0]
    g = t // TPB
    xb = x_ref[...]
    v = jnp.dot(xb, wv_ref[...], preferred_element_type=jnp.float32) + bv_ref[...]
    vr = v.reshape(g, 128, 128)
    attn = jax.lax.dot_general(
        bd_ref[...], vr, (((2,), (1,)), ((0,), (0,))),
        preferred_element_type=jnp.float32)
    attn = attn.reshape(t, HID)
    o_ref[...] = (jnp.dot(attn, wo_ref[...], preferred_element_type=jnp.float32)
                  + bo_ref[...])


def _sc_mask_softmax(sfull, topk):
    gtot = sfull.shape[0]
    info = plsc.get_sparse_core_info()
    nc, ns = info.num_cores, info.num_subcores
    nw = nc * ns
    g_per_w = gtot // nw
    mesh = plsc.VectorSubcoreMesh(core_axis_name="c", subcore_axis_name="s")

    @functools.partial(
        pl.kernel, mesh=mesh,
        out_type=jax.ShapeDtypeStruct((gtot, 128, 128), jnp.float32),
        scratch_types=[
            pltpu.VMEM((128, 128), jnp.float32),
            pltpu.VMEM((128, 128), jnp.float32),
            pltpu.SemaphoreType.DMA,
        ],
        compiler_params=pltpu.CompilerParams(needs_layout_passes=False),
    )
    def sc_kernel(sf_hbm, out_hbm, in_tile, out_tile, sem):
        wid = lax.axis_index("s") * nc + lax.axis_index("c")
        base = wid * g_per_w
        zeros16 = jnp.zeros((NHEAD,), jnp.float32)

        @pl.loop(0, 128)
        def _zero_row(r):
            @pl.loop(0, 128, step=NHEAD)
            def _zero_chunk(c):
                out_tile.at[r].at[pl.ds(c, NHEAD)][...] = zeros16

        @pl.loop(0, g_per_w)
        def _per_tile(gi):
            g = base + gi
            pltpu.async_copy(sf_hbm.at[g], in_tile, sem).wait()
            for a in range(TPB):
                @pl.loop(0, NHEAD)
                def _per_row(h, a=a):
                    s = in_tile.at[NHEAD * a + h].at[pl.ds(NHEAD * a, NHEAD)][...]
                    srt = lax.sort(s, dimension=0)
                    pos = lax.iota(jnp.int32, NHEAD)
                    thr = jnp.max(jnp.where(pos == NHEAD - topk, srt,
                                            -jnp.inf))
                    gt = s > thr
                    eq = s == thr
                    cnt_gt = jnp.sum(jnp.where(gt, 1.0, 0.0))
                    eqf = jnp.where(eq, 1.0, 0.0)
                    prefix_eq = jnp.cumsum(eqf) - eqf
                    keep = gt | (eq & (prefix_eq < (topk - cnt_gt)))
                    sp = jnp.where(keep, s, 0.0)
                    m = jnp.max(sp)
                    e = jnp.exp(sp - m)
                    w = e / jnp.sum(e)
                    out_tile.at[NHEAD * a + h].at[pl.ds(NHEAD * a, NHEAD)][...] = w

            pltpu.sync_copy(out_tile, out_hbm.at[g])

    return sc_kernel(sfull)


def _k1_scores(xf, Wq, bq2, Wk, bk2, t1, csz, c):
    n, d = xf.shape
    off = c * (csz // t1)
    return pl.pallas_call(
        _scores_kernel,
        grid=(csz // t1,),
        in_specs=[
            pl.BlockSpec((t1, d), lambda i: (i + off, 0)),
            pl.BlockSpec((d, d), lambda i: (0, 0)),
            pl.BlockSpec((1, d), lambda i: (0, 0)),
            pl.BlockSpec((d, d), lambda i: (0, 0)),
            pl.BlockSpec((1, d), lambda i: (0, 0)),
        ],
        out_specs=pl.BlockSpec((t1 // TPB, 128, 128), lambda i: (i, 0, 0)),
        out_shape=jax.ShapeDtypeStruct((csz // TPB, 128, 128), jnp.float32),
        compiler_params=pltpu.CompilerParams(
            dimension_semantics=("arbitrary",)),
    )(xf, Wq, bq2, Wk, bk2)


def _k2_output(obuf, xf, Wv, bv2, Wo, bo2, bdw, t2, csz, c):
    n, d = xf.shape
    off = c * (csz // t2)
    specs = [
        pl.BlockSpec((t2, d), lambda i: (i + off, 0)),
        pl.BlockSpec((d, d), lambda i: (0, 0)),
        pl.BlockSpec((1, d), lambda i: (0, 0)),
        pl.BlockSpec((d, d), lambda i: (0, 0)),
        pl.BlockSpec((1, d), lambda i: (0, 0)),
        pl.BlockSpec((t2 // TPB, 128, 128), lambda i: (i, 0, 0)),
    ]
    args = [xf, Wv, bv2, Wo, bo2, bdw]
    aliases = {}
    if obuf is not None:
        specs = [pl.BlockSpec(memory_space=pl.ANY)] + specs
        args = [obuf] + args
        aliases = {0: 0}
    return pl.pallas_call(
        _av_out_kernel,
        grid=(csz // t2,),
        in_specs=specs,
        out_specs=pl.BlockSpec((t2, d), lambda i: (i + off, 0)),
        out_shape=jax.ShapeDtypeStruct((n, d), jnp.float32),
        input_output_aliases=aliases,
        compiler_params=pltpu.CompilerParams(
            dimension_semantics=("arbitrary",)),
    )(*args)


@functools.partial(jax.jit, static_argnames=())
def kernel(x, Wq, bq, Wk, bk, Wv, bv, Wo, bo):
    b, s, d = x.shape
    n = b * s
    xf = x.reshape(n, d)
    topk = int(s * (1.0 - SPARSITY))
    t1 = 512
    t2 = 256
    nchunks = 2
    csz = n // nchunks

    bq2, bk2 = bq.reshape(1, d), bk.reshape(1, d)
    bv2, bo2 = bv.reshape(1, d), bo.reshape(1, d)
    bdws = []
    for c in range(nchunks):
        sfull = _k1_scores(xf, Wq, bq2, Wk, bk2, t1, csz, c)
        bdws.append(_sc_mask_softmax(sfull, topk))
    out = None
    for c in range(nchunks):
        out = _k2_output(out, xf, Wv, bv2, Wo, bo2, bdws[c], t2, csz, c)

    return out.reshape(b, s, d)
